# baseline (speedup 1.0000x reference)
import functools

import jax
import jax.numpy as jnp
from jax import lax
from jax.experimental import pallas as pl
from jax.experimental.pallas import tpu as pltpu
from jax.experimental.pallas import tpu_sc as plsc

RC = 5.0
W = 1.0
N_TYPES = 2

_NC = 2
_NS = 16
_L = 16


@functools.lru_cache(maxsize=None)
def _make_sc_bond(n_nodes: int, n_edges: int, base: int = 0, count: int | None = None):
    if count is None:
        count = n_edges
    nw = _NC * _NS
    ch = 6400
    assert count % ch == 0 and base % 128 == 0
    n_chunks = count // ch
    max_k = -(-n_chunks // nw)
    mesh = plsc.VectorSubcoreMesh(core_axis_name="c", subcore_axis_name="s")

    @functools.partial(
        pl.kernel,
        mesh=mesh,
        out_type=jax.ShapeDtypeStruct((count,), jnp.int32),
        compiler_params=pltpu.CompilerParams(needs_layout_passes=False),
        scratch_types=[
            pltpu.VMEM((((n_nodes + 127) // 128) * 128,), jnp.int32),
            pltpu.VMEM((2, 2, ch), jnp.int32),
            pltpu.VMEM((2, ch), jnp.int32),
            pltpu.SemaphoreType.DMA,
            pltpu.SemaphoreType.DMA,
            pltpu.SemaphoreType.DMA,
            pltpu.SemaphoreType.DMA,
        ],
    )
    def sc_bond(an_hbm, ei_hbm, out_hbm, an_v, in_v, bo_v, si0, si1, so0, so1):
        wid = lax.axis_index("s") * _NC + lax.axis_index("c")
        sems_in = (si0, si1)
        sems_out = (so0, so1)
        pltpu.sync_copy(an_hbm, an_v.at[pl.ds(0, n_nodes)])

        def start_in(k):
            c = wid + nw * k

            @pl.when(c < n_chunks)
            def _():
                pltpu.async_copy(ei_hbm.at[:, pl.ds(base + c * ch, ch)],
                                 in_v.at[k % 2], sems_in[k % 2])

        start_in(0)
        for k in range(max_k):
            buf = k % 2
            c = wid + nw * k
            if k + 1 < max_k:
                start_in(k + 1)

            @pl.when(c < n_chunks)
            def _():
                pltpu.make_async_copy(ei_hbm.at[:, pl.ds(base + c * ch, ch)],
                                      in_v.at[buf], sems_in[buf]).wait()
                if k >= 2:
                    pltpu.make_async_copy(
                        bo_v.at[buf],
                        out_hbm.at[pl.ds((c - 2 * nw) * ch, ch)],
                        sems_out[buf]).wait()

                @plsc.parallel_loop(0, ch, step=_L, unroll=8)
                def body(i):
                    sl = pl.ds(i, _L)
                    zi = plsc.load_gather(an_v, [in_v[buf, 0, sl]])
                    zj = plsc.load_gather(an_v, [in_v[buf, 1, sl]])
                    bo_v[buf, sl] = zi * N_TYPES + zj
                pltpu.async_copy(bo_v.at[buf],
                                 out_hbm.at[pl.ds(c * ch, ch)], sems_out[buf])

        for k in range(max(0, max_k - 3), max_k):
            buf = k % 2
            c = wid + nw * k

            @pl.when((c < n_chunks) & (c + 2 * nw >= n_chunks))
            def _():
                pltpu.make_async_copy(bo_v.at[buf],
                                      out_hbm.at[pl.ds(c * ch, ch)],
                                      sems_out[buf]).wait()

    return sc_bond


def _tc_edge_body(rij_ref, bond_ref, a1_ref, a2_ref, bl_ref, out_ref):
    be = rij_ref.shape[0]
    rij = rij_ref[...].reshape(1, be)
    b = bond_ref[...].reshape(1, be)
    rows = lax.broadcasted_iota(jnp.int32, (8, be), 0)
    w_oh = (rows == b).astype(jnp.float32)
    fcut = 1.0 / (1.0 + jnp.exp((rij - RC + 5.0 * W) / W))
    a1f = jnp.dot(a1_ref[...], w_oh * fcut,
                  preferred_element_type=jnp.float32)
    a2p = jnp.dot(jnp.abs(a2_ref[...]) + 1.0, w_oh,
                  preferred_element_type=jnp.float32)
    bl0 = bl_ref[0, 0]
    bl1 = bl_ref[0, 1]
    s = ((b >> 1) + (b & 1)).astype(jnp.float32)
    r0 = bl0 + 0.5 * s * (bl1 - bl0)
    x = jnp.log(r0 / rij)
    out_t = a1f * jnp.exp(a2p * x)
    out_ref[...] = out_t[:out_ref.shape[0], :]


def _tc_edge_body_alias(rij_ref, bond_ref, a1_ref, a2_ref, bl_ref, prev_ref,
                        out_ref):
    del prev_ref
    _tc_edge_body(rij_ref, bond_ref, a1_ref, a2_ref, bl_ref, out_ref)


def _tc_node_body(z_ref, o_ref, out_ref):
    bn = z_ref.shape[0]
    z = z_ref[...].reshape(1, bn).astype(jnp.float32)
    c0 = o_ref[:, 0:1]
    c1 = o_ref[:, 1:2]
    nf_t = c0 + z * (c1 - c0)
    out_ref[...] = nf_t[:out_ref.shape[0], :]


def kernel(atomic_numbers, edge_index, edge_length, hopping_param,
           onsite_param, bond_length_list):
    n_nodes = atomic_numbers.shape[0]
    n_edges = edge_index.shape[1]
    edge_me = hopping_param.shape[1]
    node_me = onsite_param.shape[1]

    an = atomic_numbers.astype(jnp.int32)
    ei = edge_index.astype(jnp.int32)

    be = 102400
    nblk_a = 8
    split = nblk_a * be
    bond_a = _make_sc_bond(n_nodes, n_edges, 0, split)(an, ei)
    bond_b = _make_sc_bond(n_nodes, n_edges, split, n_edges - split)(an, ei)

    a1t = jnp.zeros((16, 8), jnp.float32).at[:edge_me, :4].set(
        hopping_param[:, :, 0].T)
    a2t = jnp.zeros((16, 8), jnp.float32).at[:edge_me, :4].set(
        hopping_param[:, :, 1].T)
    ot = jnp.zeros((8, 2), jnp.float32).at[:node_me, :].set(
        onsite_param[:, :, 0].T)

    bl2 = bond_length_list.reshape(1, 2)
    ef_a = pl.pallas_call(
        _tc_edge_body,
        grid=(nblk_a,),
        in_specs=[
            pl.BlockSpec((be,), lambda i: (i,)),
            pl.BlockSpec((be,), lambda i: (i,)),
            pl.BlockSpec((16, 8), lambda i: (0, 0)),
            pl.BlockSpec((16, 8), lambda i: (0, 0)),
            pl.BlockSpec((1, 2), lambda i: (0, 0)),
        ],
        out_specs=pl.BlockSpec((edge_me, be), lambda i: (0, i)),
        out_shape=jax.ShapeDtypeStruct((edge_me, n_edges), jnp.float32),
    )(edge_length, bond_a, a1t, a2t, bl2)

    grid_b = pl.cdiv(n_edges - split, be)
    ef_t = pl.pallas_call(
        _tc_edge_body_alias,
        grid=(grid_b,),
        in_specs=[
            pl.BlockSpec((be,), lambda i: (i + nblk_a,)),
            pl.BlockSpec((be,), lambda i: (i,)),
            pl.BlockSpec((16, 8), lambda i: (0, 0)),
            pl.BlockSpec((16, 8), lambda i: (0, 0)),
            pl.BlockSpec((1, 2), lambda i: (0, 0)),
            pl.BlockSpec(memory_space=pltpu.MemorySpace.HBM),
        ],
        out_specs=pl.BlockSpec((edge_me, be), lambda i: (0, i + nblk_a)),
        out_shape=jax.ShapeDtypeStruct((edge_me, n_edges), jnp.float32),
        input_output_aliases={5: 0},
    )(edge_length, bond_b, a1t, a2t, bl2, ef_a)

    nf_t = pl.pallas_call(
        _tc_node_body,
        in_specs=[
            pl.BlockSpec((n_nodes,), lambda: (0,)),
            pl.BlockSpec((8, 2), lambda: (0, 0)),
        ],
        out_specs=pl.BlockSpec((node_me, n_nodes), lambda: (0, 0)),
        out_shape=jax.ShapeDtypeStruct((node_me, n_nodes), jnp.float32),
    )(an, ot)

    return ef_t.T, nf_t.T
